# baseline (speedup 1.0000x reference)
import functools

import jax
import jax.numpy as jnp
from jax import lax
from jax.experimental import pallas as pl
from jax.experimental.pallas import tpu as pltpu
from jax.experimental.pallas import tpu_sc as plsc

NUM_NODES = 30
NUM_IN = 2
GCN_OUT = 64
HID = 512
T = 128
S_IN = 5
S_OUT = 3
N_TOTAL = T * NUM_NODES
N_EDGES = 32768

NN_PAD = 11776
SLICE = NN_PAD // 16
E_CORE = 3 * N_EDGES
E_TILE = E_CORE // 16
CHUNK = 128
NCHUNK = E_TILE // CHUNK
DUMMY_DST = 11520
NODE_PAD = 32


TBL = 32776


def _sc_body(ei_hbm, x0_hbm, x1_hbm, tbl_hbm, out0_hbm, out1_hbm,
             src_v, dst_v, x0_v, x1_v, vals0_v, vals1_v, vals0b_v, vals1b_v,
             degs_v, dinvs_v, s0_v, s1_v, ones_v, table_v,
             sem_a0, sem_a1, sem_b0, sem_b1,
             deg_sh, xs0_sh, xs1_sh, out0_sh, out1_sh):
    c = lax.axis_index("c")
    s = lax.axis_index("s")
    base = s * SLICE
    nch = jnp.where(c == 0, NCHUNK, (2 * N_EDGES) // (16 * CHUNK))

    for sl in range(3):
        def _stage(sl=sl):
            st = 3 * c + sl
            pltpu.sync_copy(ei_hbm.at[st, 0, pl.ds(s * 16, 16)],
                            src_v.at[pl.ds(sl * 16, 16)])
            pltpu.sync_copy(ei_hbm.at[st, 1, pl.ds(s * 16, 16)],
                            dst_v.at[pl.ds(sl * 16, 16)])
        if sl < 2:
            _stage()
        else:
            pl.when(c == 0)(_stage)
    for sl in (1, 2):
        off = jnp.full((16,), sl * N_TOTAL, jnp.int32)
        for r in range(16):
            for k in range(CHUNK // 16):
                row = sl * 16 + r
                src_v[row, pl.ds(k * 16, 16)] = (
                    src_v[row, pl.ds(k * 16, 16)] + off)
                dst_v[row, pl.ds(k * 16, 16)] = (
                    dst_v[row, pl.ds(k * 16, 16)] + off)
    pltpu.sync_copy(x0_hbm.at[pl.ds(c * NN_PAD + base, SLICE)], s0_v)
    pltpu.sync_copy(x1_hbm.at[pl.ds(c * NN_PAD + base, SLICE)], s1_v)
    pltpu.sync_copy(tbl_hbm, table_v)

    for i in range(CHUNK // 16):
        ones_v[pl.ds(i * 16, 16)] = jnp.full((16,), 1.0, jnp.float32)
    for i in range(SLICE // 16):
        degs_v[pl.ds(i * 16, 16)] = jnp.full((16,), 0.0, jnp.float32)
    pltpu.sync_copy(degs_v, deg_sh.at[pl.ds(base, SLICE)])
    plsc.subcore_barrier()

    def deg_step(j, carry):
        @pl.when(j % 2 == 0)
        def _even():
            @pl.when(j >= 2)
            def _w():
                pltpu.make_async_copy(ones_v, deg_sh.at[dst_v.at[j]],
                                      sem_a0).wait()
            pltpu.async_copy(ones_v, deg_sh.at[dst_v.at[j]], sem_a0,
                             add=True)

        @pl.when(j % 2 == 1)
        def _odd():
            @pl.when(j >= 2)
            def _w():
                pltpu.make_async_copy(ones_v, deg_sh.at[dst_v.at[j]],
                                      sem_b0).wait()
            pltpu.async_copy(ones_v, deg_sh.at[dst_v.at[j]], sem_b0,
                             add=True)
        return carry
    lax.fori_loop(0, nch, deg_step, 0)
    pltpu.make_async_copy(ones_v, deg_sh.at[dst_v.at[0]], sem_a0).wait()
    pltpu.make_async_copy(ones_v, deg_sh.at[dst_v.at[0]], sem_b0).wait()
    plsc.subcore_barrier()

    pltpu.sync_copy(deg_sh.at[pl.ds(base, SLICE)], degs_v)
    for i in range(SLICE // 16):
        d = degs_v[pl.ds(i * 16, 16)] + 1.0
        y = plsc.load_gather(table_v, [d.astype(jnp.int32)])
        dinvs_v[pl.ds(i * 16, 16)] = y
        s0_v[pl.ds(i * 16, 16)] = s0_v[pl.ds(i * 16, 16)] * y
        s1_v[pl.ds(i * 16, 16)] = s1_v[pl.ds(i * 16, 16)] * y
    pltpu.sync_copy(s0_v, xs0_sh.at[pl.ds(base, SLICE)])
    pltpu.sync_copy(s1_v, xs1_sh.at[pl.ds(base, SLICE)])
    pltpu.sync_copy(s0_v, out0_sh.at[pl.ds(base, SLICE)])
    pltpu.sync_copy(s1_v, out1_sh.at[pl.ds(base, SLICE)])
    plsc.subcore_barrier()

    pltpu.sync_copy(xs0_sh, x0_v)
    pltpu.sync_copy(xs1_sh, x1_v)

    def edge_step(j, carry):
        def run(b0, b1, s0, s1):
            @pl.when(j >= 2)
            def _w():
                pltpu.make_async_copy(b0, out0_sh.at[dst_v.at[j]], s0).wait()
                pltpu.make_async_copy(b1, out1_sh.at[dst_v.at[j]], s1).wait()
            for k in range(CHUNK // 16):
                src = src_v[j, pl.ds(k * 16, 16)]
                b0[pl.ds(k * 16, 16)] = plsc.load_gather(x0_v, [src])
                b1[pl.ds(k * 16, 16)] = plsc.load_gather(x1_v, [src])
            pltpu.async_copy(b0, out0_sh.at[dst_v.at[j]], s0, add=True)
            pltpu.async_copy(b1, out1_sh.at[dst_v.at[j]], s1, add=True)

        @pl.when(j % 2 == 0)
        def _even():
            run(vals0_v, vals1_v, sem_a0, sem_a1)

        @pl.when(j % 2 == 1)
        def _odd():
            run(vals0b_v, vals1b_v, sem_b0, sem_b1)
        return carry
    lax.fori_loop(0, nch, edge_step, 0)
    pltpu.make_async_copy(vals0_v, out0_sh.at[dst_v.at[0]], sem_a0).wait()
    pltpu.make_async_copy(vals1_v, out1_sh.at[dst_v.at[0]], sem_a1).wait()
    pltpu.make_async_copy(vals0b_v, out0_sh.at[dst_v.at[0]], sem_b0).wait()
    pltpu.make_async_copy(vals1b_v, out1_sh.at[dst_v.at[0]], sem_b1).wait()
    plsc.subcore_barrier()

    pltpu.sync_copy(out0_sh.at[pl.ds(base, SLICE)], s0_v)
    pltpu.sync_copy(out1_sh.at[pl.ds(base, SLICE)], s1_v)
    for i in range(SLICE // 16):
        y = dinvs_v[pl.ds(i * 16, 16)]
        s0_v[pl.ds(i * 16, 16)] = s0_v[pl.ds(i * 16, 16)] * y
        s1_v[pl.ds(i * 16, 16)] = s1_v[pl.ds(i * 16, 16)] * y
    pltpu.sync_copy(s0_v, out0_hbm.at[pl.ds(c * NN_PAD + base, SLICE)])
    pltpu.sync_copy(s1_v, out1_hbm.at[pl.ds(c * NN_PAD + base, SLICE)])


@functools.cache
def _sc_scatter_kernel():
    return functools.partial(
        pl.kernel,
        out_type=[jax.ShapeDtypeStruct((2 * NN_PAD,), jnp.float32),
                  jax.ShapeDtypeStruct((2 * NN_PAD,), jnp.float32)],
        mesh=plsc.VectorSubcoreMesh(core_axis_name="c", subcore_axis_name="s",
                                    num_cores=2, num_subcores=16),
        compiler_params=pltpu.CompilerParams(needs_layout_passes=False),
        scratch_types=[
        pltpu.VMEM((NCHUNK, CHUNK), jnp.int32),
        pltpu.VMEM((NCHUNK, CHUNK), jnp.int32),
        pltpu.VMEM((NN_PAD,), jnp.float32),
        pltpu.VMEM((NN_PAD,), jnp.float32),
        pltpu.VMEM((CHUNK,), jnp.float32),
        pltpu.VMEM((CHUNK,), jnp.float32),
        pltpu.VMEM((CHUNK,), jnp.float32),
        pltpu.VMEM((CHUNK,), jnp.float32),
        pltpu.VMEM((SLICE,), jnp.float32),
        pltpu.VMEM((SLICE,), jnp.float32),
        pltpu.VMEM((SLICE,), jnp.float32),
        pltpu.VMEM((SLICE,), jnp.float32),
        pltpu.VMEM((CHUNK,), jnp.float32),
        pltpu.VMEM((TBL,), jnp.float32),
        pltpu.SemaphoreType.DMA,
        pltpu.SemaphoreType.DMA,
        pltpu.SemaphoreType.DMA,
        pltpu.SemaphoreType.DMA,
        pltpu.VMEM_SHARED((NN_PAD,), jnp.float32),
        pltpu.VMEM_SHARED((NN_PAD,), jnp.float32),
        pltpu.VMEM_SHARED((NN_PAD,), jnp.float32),
        pltpu.VMEM_SHARED((NN_PAD,), jnp.float32),
        pltpu.VMEM_SHARED((NN_PAD,), jnp.float32),
        ],
    )(_sc_body)


def _fold_body(p_ref, w_ref, u_ref):
    u_ref[...] = jnp.dot(p_ref[...], w_ref[...],
                         preferred_element_type=jnp.float32)


def _fold_u(P, W_ih):
    n_blk = 8
    blk = (4 * HID) // n_blk
    return pl.pallas_call(
        _fold_body,
        grid=(n_blk,),
        in_specs=[
            pl.BlockSpec((72, GCN_OUT * NUM_NODES), lambda n: (0, 0)),
            pl.BlockSpec((GCN_OUT * NUM_NODES, blk), lambda n: (0, n)),
        ],
        out_specs=pl.BlockSpec((72, blk), lambda n: (0, n)),
        out_shape=jax.ShapeDtypeStruct((72, 4 * HID), jnp.float32),
    )(P, W_ih)


def _seq_body(fi0_ref, o0_ref, o1_ref, u_ref, whh_ref, bsum_ref,
              w1_ref, b1_ref, w2_ref, b2_ref, w3_ref, b3_ref,
              w4_ref, b4_ref, sx_ref, sy_ref, stats_ref, enc_ref, dec_ref):
    enc_ref[0] = fi0_ref[...]
    U0 = u_ref[0:32, :]
    U1 = u_ref[32:64, :]
    bvec = u_ref[64:65, :]
    bias = bvec + bsum_ref[...]
    std0 = stats_ref[0:1, 0:1]
    std1 = stats_ref[0:1, 1:2]
    mean0 = stats_ref[1:2, 0:1]
    mean1 = stats_ref[1:2, 1:2]

    h = jnp.zeros((T, HID), jnp.float32)
    c = jnp.zeros((T, HID), jnp.float32)
    p = None
    for step in range(S_IN - 1 + S_OUT):
        if step < S_IN:
            m0 = o0_ref[step]
            m1 = o1_ref[step]
        else:
            pxs = jnp.dot(p, sx_ref[...], preferred_element_type=jnp.float32)
            pys = jnp.dot(p, sy_ref[...], preferred_element_type=jnp.float32)
            fx = pxs * std0 + mean0
            fy = pys * std1 + mean1
            col = lax.broadcasted_iota(jnp.int32, (T, NODE_PAD), 1)
            exn = jnp.where((fx > 0.04) & (fy > 0.04) & (col < NUM_NODES),
                            1.0, 0.0)
            dx = fx[:, :, None] - fx[:, None, :]
            dy = fy[:, :, None] - fy[:, None, :]
            d2 = dx * dx + dy * dy
            cond = jnp.where((d2 > 0.0) & (d2 < 100.0), 1.0, 0.0)
            ep = exn[:, :, None] * exn[:, None, :]
            r = lax.broadcasted_iota(jnp.int32, (T, NODE_PAD, NODE_PAD), 1)
            q = lax.broadcasted_iota(jnp.int32, (T, NODE_PAD, NODE_PAD), 2)
            eye = r == q
            A = jnp.where(eye, 1.0, ep * cond)
            deg = jnp.sum(A, axis=-1)
            dinv = lax.rsqrt(jnp.maximum(deg, 1e-12))
            An = A * (dinv[:, :, None] * dinv[:, None, :])
            m0 = jnp.sum(An * pxs[:, None, :], axis=-1)
            m1 = jnp.sum(An * pys[:, None, :], axis=-1)
        gates = (jnp.dot(m0, U0, preferred_element_type=jnp.float32)
                 + jnp.dot(m1, U1, preferred_element_type=jnp.float32)
                 + jnp.dot(h, whh_ref[...], preferred_element_type=jnp.float32)
                 + bias)
        i_g = jax.nn.sigmoid(gates[:, 0:HID])
        f_g = jax.nn.sigmoid(gates[:, HID:2 * HID])
        g_g = jnp.tanh(gates[:, 2 * HID:3 * HID])
        o_g = jax.nn.sigmoid(gates[:, 3 * HID:4 * HID])
        c = f_g * c + i_g * g_g
        h = o_g * jnp.tanh(c)
        m = jax.nn.relu(jnp.dot(h, w1_ref[...],
                                preferred_element_type=jnp.float32)
                        + b1_ref[...])
        m = jax.nn.relu(jnp.dot(m, w2_ref[...],
                                preferred_element_type=jnp.float32)
                        + b2_ref[...])
        m = jax.nn.relu(jnp.dot(m, w3_ref[...],
                                preferred_element_type=jnp.float32)
                        + b3_ref[...])
        p = jnp.dot(m, w4_ref[...],
                    preferred_element_type=jnp.float32) + b4_ref[...]
        if step < S_IN - 1:
            enc_ref[step + 1] = p
        else:
            dec_ref[step - (S_IN - 1)] = p


def _seq_run(fi0, o0p, o1p, U, W_hh, bsum, W1, b1, W2, b2, W3, b3, W4, b4,
             Sx, Sy, stats):
    return pl.pallas_call(
        _seq_body,
        out_shape=[
            jax.ShapeDtypeStruct((S_IN, T, NUM_IN * NUM_NODES), jnp.float32),
            jax.ShapeDtypeStruct((S_OUT, T, NUM_IN * NUM_NODES), jnp.float32),
        ],
    )(fi0, o0p, o1p, U, W_hh, bsum, W1, b1, W2, b2, W3, b3, W4, b4,
      Sx, Sy, stats)


def kernel(feature_input, edge_index, batch_index, number_of_trajectories,
           stats, gcn_W, gcn_b, W_ih, W_hh, b_ih, b_hh,
           W1, b1, W2, b2, W3, b3, W4, b4):
    ei4 = edge_index.astype(jnp.int32).reshape(S_IN, 2, N_EDGES // CHUNK,
                                               CHUNK)

    xf = feature_input.reshape(S_IN * N_TOTAL, NUM_IN)
    pad0 = NN_PAD - 3 * N_TOTAL
    pad1 = NN_PAD - 2 * N_TOTAL
    x0_in = jnp.concatenate([
        jnp.pad(xf[:3 * N_TOTAL, 0], (0, pad0)),
        jnp.pad(xf[3 * N_TOTAL:, 0], (0, pad1)),
    ])
    x1_in = jnp.concatenate([
        jnp.pad(xf[:3 * N_TOTAL, 1], (0, pad0)),
        jnp.pad(xf[3 * N_TOTAL:, 1], (0, pad1)),
    ])

    rsqrt_tbl = lax.rsqrt(jnp.maximum(
        jnp.arange(TBL, dtype=jnp.float32), 1.0))
    out0, out1 = _sc_scatter_kernel()(ei4, x0_in, x1_in, rsqrt_tbl)
    o0 = jnp.concatenate([out0[:3 * N_TOTAL],
                          out0[NN_PAD:NN_PAD + 2 * N_TOTAL]])
    o1 = jnp.concatenate([out1[:3 * N_TOTAL],
                          out1[NN_PAD:NN_PAD + 2 * N_TOTAL]])
    o0p = jnp.pad(o0.reshape(S_IN, T, NUM_NODES), ((0, 0), (0, 0), (0, 2)))
    o1p = jnp.pad(o1.reshape(S_IN, T, NUM_NODES), ((0, 0), (0, 0), (0, 2)))

    K = GCN_OUT * NUM_NODES
    eye30 = jnp.eye(NUM_NODES, dtype=jnp.float32)
    P0 = jnp.kron(eye30, gcn_W[0:1, :])
    P1 = jnp.kron(eye30, gcn_W[1:2, :])
    bb = jnp.tile(gcn_b, NUM_NODES)[None, :]
    zrow2 = jnp.zeros((2, K), jnp.float32)
    zrow7 = jnp.zeros((7, K), jnp.float32)
    P = jnp.concatenate([P0, zrow2, P1, zrow2, bb, zrow7])
    U = _fold_u(P, W_ih)

    bsum = (b_ih + b_hh)[None, :]
    k60 = jnp.arange(NUM_IN * NUM_NODES)[:, None]
    n32 = jnp.arange(NODE_PAD)[None, :]
    Sx = ((k60 == 2 * n32) & (n32 < NUM_NODES)).astype(jnp.float32)
    Sy = ((k60 == 2 * n32 + 1) & (n32 < NUM_NODES)).astype(jnp.float32)

    fi0 = feature_input[0].reshape(T, NUM_IN * NUM_NODES)
    enc_o, dec_o = _seq_run(fi0, o0p, o1p, U, W_hh, bsum,
                            W1, b1[None, :], W2, b2[None, :], W3, b3[None, :],
                            W4, b4[None, :], Sx, Sy, stats)

    enc = enc_o.reshape(S_IN, N_TOTAL, NUM_IN)
    dec = dec_o.reshape(S_OUT, N_TOTAL, NUM_IN)
    return enc, dec


# speedup vs baseline: 1.0652x; 1.0652x over previous
import functools

import jax
import jax.numpy as jnp
from jax import lax
from jax.experimental import pallas as pl
from jax.experimental.pallas import tpu as pltpu
from jax.experimental.pallas import tpu_sc as plsc

NUM_NODES = 30
NUM_IN = 2
GCN_OUT = 64
HID = 512
T = 128
S_IN = 5
S_OUT = 3
N_TOTAL = T * NUM_NODES
N_EDGES = 32768

NN_PAD = 11776
SLICE = NN_PAD // 16
E_CORE = 3 * N_EDGES
E_TILE = E_CORE // 16
CHUNK = 128
NCHUNK = E_TILE // CHUNK
DUMMY_DST = 11520
NODE_PAD = 32


TBL = 32776


def _sc_body(ei_hbm, x0_hbm, x1_hbm, tbl_hbm, out0_hbm, out1_hbm,
             src_v, dst_v, x0_v, x1_v, vals0_v, vals1_v, vals0b_v, vals1b_v,
             degs_v, dinvs_v, s0_v, s1_v, ones_v, table_v,
             sem_a0, sem_a1, sem_b0, sem_b1, sem_c,
             deg_sh, xs0_sh, xs1_sh, out0_sh, out1_sh):
    c = lax.axis_index("c")
    s = lax.axis_index("s")
    base = s * SLICE
    nch = jnp.where(c == 0, NCHUNK, (2 * N_EDGES) // (16 * CHUNK))

    for sl in range(3):
        def _stage(sl=sl):
            st = 3 * c + sl
            pltpu.sync_copy(ei_hbm.at[st, 0, pl.ds(s * 16, 16)],
                            src_v.at[pl.ds(sl * 16, 16)])
            pltpu.sync_copy(ei_hbm.at[st, 1, pl.ds(s * 16, 16)],
                            dst_v.at[pl.ds(sl * 16, 16)])
        if sl < 2:
            _stage()
        else:
            pl.when(c == 0)(_stage)
    for sl in (1, 2):
        off = jnp.full((16,), sl * N_TOTAL, jnp.int32)
        for r in range(16):
            for k in range(CHUNK // 16):
                row = sl * 16 + r
                src_v[row, pl.ds(k * 16, 16)] = (
                    src_v[row, pl.ds(k * 16, 16)] + off)
                dst_v[row, pl.ds(k * 16, 16)] = (
                    dst_v[row, pl.ds(k * 16, 16)] + off)
    pltpu.async_copy(x0_hbm.at[pl.ds(c * NN_PAD + base, SLICE)], s0_v, sem_a1)
    pltpu.async_copy(x1_hbm.at[pl.ds(c * NN_PAD + base, SLICE)], s1_v, sem_b1)
    pltpu.async_copy(tbl_hbm, table_v, sem_c)

    for i in range(CHUNK // 16):
        ones_v[pl.ds(i * 16, 16)] = jnp.full((16,), 1.0, jnp.float32)
    for i in range(SLICE // 16):
        degs_v[pl.ds(i * 16, 16)] = jnp.full((16,), 0.0, jnp.float32)
    pltpu.sync_copy(degs_v, deg_sh.at[pl.ds(base, SLICE)])
    plsc.subcore_barrier()

    def deg_step(j, carry):
        @pl.when(j % 2 == 0)
        def _even():
            @pl.when(j >= 2)
            def _w():
                pltpu.make_async_copy(ones_v, deg_sh.at[dst_v.at[j]],
                                      sem_a0).wait()
            pltpu.async_copy(ones_v, deg_sh.at[dst_v.at[j]], sem_a0,
                             add=True)

        @pl.when(j % 2 == 1)
        def _odd():
            @pl.when(j >= 2)
            def _w():
                pltpu.make_async_copy(ones_v, deg_sh.at[dst_v.at[j]],
                                      sem_b0).wait()
            pltpu.async_copy(ones_v, deg_sh.at[dst_v.at[j]], sem_b0,
                             add=True)
        return carry
    lax.fori_loop(0, nch, deg_step, 0)
    pltpu.make_async_copy(ones_v, deg_sh.at[dst_v.at[0]], sem_a0).wait()
    pltpu.make_async_copy(ones_v, deg_sh.at[dst_v.at[0]], sem_b0).wait()
    plsc.subcore_barrier()

    pltpu.make_async_copy(x0_hbm.at[pl.ds(c * NN_PAD + base, SLICE)],
                          s0_v, sem_a1).wait()
    pltpu.make_async_copy(x1_hbm.at[pl.ds(c * NN_PAD + base, SLICE)],
                          s1_v, sem_b1).wait()
    pltpu.make_async_copy(tbl_hbm, table_v, sem_c).wait()
    pltpu.sync_copy(deg_sh.at[pl.ds(base, SLICE)], degs_v)
    for i in range(SLICE // 16):
        d = degs_v[pl.ds(i * 16, 16)] + 1.0
        y = plsc.load_gather(table_v, [d.astype(jnp.int32)])
        dinvs_v[pl.ds(i * 16, 16)] = y
        s0_v[pl.ds(i * 16, 16)] = s0_v[pl.ds(i * 16, 16)] * y
        s1_v[pl.ds(i * 16, 16)] = s1_v[pl.ds(i * 16, 16)] * y
    pltpu.sync_copy(s0_v, xs0_sh.at[pl.ds(base, SLICE)])
    pltpu.sync_copy(s1_v, xs1_sh.at[pl.ds(base, SLICE)])
    pltpu.sync_copy(s0_v, out0_sh.at[pl.ds(base, SLICE)])
    pltpu.sync_copy(s1_v, out1_sh.at[pl.ds(base, SLICE)])
    plsc.subcore_barrier()

    pltpu.sync_copy(xs0_sh, x0_v)
    pltpu.sync_copy(xs1_sh, x1_v)

    def edge_step(j, carry):
        def run(b0, b1, s0, s1):
            @pl.when(j >= 2)
            def _w():
                pltpu.make_async_copy(b0, out0_sh.at[dst_v.at[j]], s0).wait()
                pltpu.make_async_copy(b1, out1_sh.at[dst_v.at[j]], s1).wait()
            for k in range(CHUNK // 16):
                src = src_v[j, pl.ds(k * 16, 16)]
                b0[pl.ds(k * 16, 16)] = plsc.load_gather(x0_v, [src])
                b1[pl.ds(k * 16, 16)] = plsc.load_gather(x1_v, [src])
            pltpu.async_copy(b0, out0_sh.at[dst_v.at[j]], s0, add=True)
            pltpu.async_copy(b1, out1_sh.at[dst_v.at[j]], s1, add=True)

        @pl.when(j % 2 == 0)
        def _even():
            run(vals0_v, vals1_v, sem_a0, sem_a1)

        @pl.when(j % 2 == 1)
        def _odd():
            run(vals0b_v, vals1b_v, sem_b0, sem_b1)
        return carry
    lax.fori_loop(0, nch, edge_step, 0)
    pltpu.make_async_copy(vals0_v, out0_sh.at[dst_v.at[0]], sem_a0).wait()
    pltpu.make_async_copy(vals1_v, out1_sh.at[dst_v.at[0]], sem_a1).wait()
    pltpu.make_async_copy(vals0b_v, out0_sh.at[dst_v.at[0]], sem_b0).wait()
    pltpu.make_async_copy(vals1b_v, out1_sh.at[dst_v.at[0]], sem_b1).wait()
    plsc.subcore_barrier()

    pltpu.sync_copy(out0_sh.at[pl.ds(base, SLICE)], s0_v)
    pltpu.sync_copy(out1_sh.at[pl.ds(base, SLICE)], s1_v)
    for i in range(SLICE // 16):
        y = dinvs_v[pl.ds(i * 16, 16)]
        s0_v[pl.ds(i * 16, 16)] = s0_v[pl.ds(i * 16, 16)] * y
        s1_v[pl.ds(i * 16, 16)] = s1_v[pl.ds(i * 16, 16)] * y
    pltpu.sync_copy(s0_v, out0_hbm.at[pl.ds(c * NN_PAD + base, SLICE)])
    pltpu.sync_copy(s1_v, out1_hbm.at[pl.ds(c * NN_PAD + base, SLICE)])


@functools.cache
def _sc_scatter_kernel():
    return functools.partial(
        pl.kernel,
        out_type=[jax.ShapeDtypeStruct((2 * NN_PAD,), jnp.float32),
                  jax.ShapeDtypeStruct((2 * NN_PAD,), jnp.float32)],
        mesh=plsc.VectorSubcoreMesh(core_axis_name="c", subcore_axis_name="s",
                                    num_cores=2, num_subcores=16),
        compiler_params=pltpu.CompilerParams(needs_layout_passes=False),
        scratch_types=[
        pltpu.VMEM((NCHUNK, CHUNK), jnp.int32),
        pltpu.VMEM((NCHUNK, CHUNK), jnp.int32),
        pltpu.VMEM((NN_PAD,), jnp.float32),
        pltpu.VMEM((NN_PAD,), jnp.float32),
        pltpu.VMEM((CHUNK,), jnp.float32),
        pltpu.VMEM((CHUNK,), jnp.float32),
        pltpu.VMEM((CHUNK,), jnp.float32),
        pltpu.VMEM((CHUNK,), jnp.float32),
        pltpu.VMEM((SLICE,), jnp.float32),
        pltpu.VMEM((SLICE,), jnp.float32),
        pltpu.VMEM((SLICE,), jnp.float32),
        pltpu.VMEM((SLICE,), jnp.float32),
        pltpu.VMEM((CHUNK,), jnp.float32),
        pltpu.VMEM((TBL,), jnp.float32),
        pltpu.SemaphoreType.DMA,
        pltpu.SemaphoreType.DMA,
        pltpu.SemaphoreType.DMA,
        pltpu.SemaphoreType.DMA,
        pltpu.SemaphoreType.DMA,
        pltpu.VMEM_SHARED((NN_PAD,), jnp.float32),
        pltpu.VMEM_SHARED((NN_PAD,), jnp.float32),
        pltpu.VMEM_SHARED((NN_PAD,), jnp.float32),
        pltpu.VMEM_SHARED((NN_PAD,), jnp.float32),
        pltpu.VMEM_SHARED((NN_PAD,), jnp.float32),
        ],
    )(_sc_body)


def _fold_body(p_ref, w_ref, u_ref):
    u_ref[...] = jnp.dot(p_ref[...], w_ref[...],
                         preferred_element_type=jnp.float32)


def _fold_u(P, W_ih):
    n_blk = 8
    blk = (4 * HID) // n_blk
    return pl.pallas_call(
        _fold_body,
        grid=(n_blk,),
        in_specs=[
            pl.BlockSpec((72, GCN_OUT * NUM_NODES), lambda n: (0, 0)),
            pl.BlockSpec((GCN_OUT * NUM_NODES, blk), lambda n: (0, n)),
        ],
        out_specs=pl.BlockSpec((72, blk), lambda n: (0, n)),
        out_shape=jax.ShapeDtypeStruct((72, 4 * HID), jnp.float32),
    )(P, W_ih)


def _seq_body(fi0_ref, o0_ref, o1_ref, u_ref, whh_ref, bsum_ref,
              w1_ref, b1_ref, w2_ref, b2_ref, w3_ref, b3_ref,
              w4_ref, b4_ref, sx_ref, sy_ref, rm_ref, qm_ref, rt_ref,
              eyep_ref, stats_ref, enc_ref, dec_ref):
    enc_ref[0] = fi0_ref[...]
    U0 = u_ref[0:32, :]
    U1 = u_ref[32:64, :]
    bvec = u_ref[64:65, :]
    bias = bvec + bsum_ref[...]
    std0 = stats_ref[0:1, 0:1]
    std1 = stats_ref[0:1, 1:2]
    mean0 = stats_ref[1:2, 0:1]
    mean1 = stats_ref[1:2, 1:2]

    h = jnp.zeros((T, HID), jnp.float32)
    c = jnp.zeros((T, HID), jnp.float32)
    p = None
    for step in range(S_IN - 1 + S_OUT):
        if step < S_IN:
            m0 = o0_ref[step]
            m1 = o1_ref[step]
        else:
            def mm(a, b):
                return jnp.dot(a, b, preferred_element_type=jnp.float32)
            pxs = mm(p, sx_ref[...])
            pys = mm(p, sy_ref[...])
            fx = pxs * std0 + mean0
            fy = pys * std1 + mean1
            col = lax.broadcasted_iota(jnp.int32, (T, NODE_PAD), 1)
            exn = jnp.where((fx > 0.04) & (fy > 0.04) & (col < NUM_NODES),
                            1.0, 0.0)
            Rm = rm_ref[...]
            Qm = qm_ref[...]
            fxr = mm(pxs, Rm) * std0 + mean0
            fxt = mm(pxs, Qm) * std0 + mean0
            fyr = mm(pys, Rm) * std1 + mean1
            fyt = mm(pys, Qm) * std1 + mean1
            dx = fxr - fxt
            dy = fyr - fyt
            d2 = dx * dx + dy * dy
            cond = jnp.where((d2 > 0.0) & (d2 < 100.0), 1.0, 0.0)
            ep = mm(exn, Rm) * mm(exn, Qm)
            eyep = eyep_ref[...]
            A = eyep + (1.0 - eyep) * (ep * cond)
            deg = mm(A, rt_ref[...])
            dinv = lax.rsqrt(jnp.maximum(deg, 1e-12))
            W = A * (mm(dinv, Rm) * mm(dinv, Qm))
            m0 = mm(W * mm(pxs, Qm), rt_ref[...])
            m1 = mm(W * mm(pys, Qm), rt_ref[...])
        gates = (jnp.dot(m0, U0, preferred_element_type=jnp.float32)
                 + jnp.dot(m1, U1, preferred_element_type=jnp.float32)
                 + jnp.dot(h, whh_ref[...], preferred_element_type=jnp.float32)
                 + bias)
        i_g = jax.nn.sigmoid(gates[:, 0:HID])
        f_g = jax.nn.sigmoid(gates[:, HID:2 * HID])
        g_g = jnp.tanh(gates[:, 2 * HID:3 * HID])
        o_g = jax.nn.sigmoid(gates[:, 3 * HID:4 * HID])
        c = f_g * c + i_g * g_g
        h = o_g * jnp.tanh(c)
        m = jax.nn.relu(jnp.dot(h, w1_ref[...],
                                preferred_element_type=jnp.float32)
                        + b1_ref[...])
        m = jax.nn.relu(jnp.dot(m, w2_ref[...],
                                preferred_element_type=jnp.float32)
                        + b2_ref[...])
        m = jax.nn.relu(jnp.dot(m, w3_ref[...],
                                preferred_element_type=jnp.float32)
                        + b3_ref[...])
        p = jnp.dot(m, w4_ref[...],
                    preferred_element_type=jnp.float32) + b4_ref[...]
        if step < S_IN - 1:
            enc_ref[step + 1] = p
        else:
            dec_ref[step - (S_IN - 1)] = p


def _seq_run(fi0, o0p, o1p, U, W_hh, bsum, W1, b1, W2, b2, W3, b3, W4, b4,
             Sx, Sy, Rm, Qm, RT, eyep, stats):
    return pl.pallas_call(
        _seq_body,
        out_shape=[
            jax.ShapeDtypeStruct((S_IN, T, NUM_IN * NUM_NODES), jnp.float32),
            jax.ShapeDtypeStruct((S_OUT, T, NUM_IN * NUM_NODES), jnp.float32),
        ],
    )(fi0, o0p, o1p, U, W_hh, bsum, W1, b1, W2, b2, W3, b3, W4, b4,
      Sx, Sy, Rm, Qm, RT, eyep, stats)


def kernel(feature_input, edge_index, batch_index, number_of_trajectories,
           stats, gcn_W, gcn_b, W_ih, W_hh, b_ih, b_hh,
           W1, b1, W2, b2, W3, b3, W4, b4):
    ei4 = edge_index.astype(jnp.int32).reshape(S_IN, 2, N_EDGES // CHUNK,
                                               CHUNK)

    xf = feature_input.reshape(S_IN * N_TOTAL, NUM_IN)
    pad0 = NN_PAD - 3 * N_TOTAL
    pad1 = NN_PAD - 2 * N_TOTAL
    x0_in = jnp.concatenate([
        jnp.pad(xf[:3 * N_TOTAL, 0], (0, pad0)),
        jnp.pad(xf[3 * N_TOTAL:, 0], (0, pad1)),
    ])
    x1_in = jnp.concatenate([
        jnp.pad(xf[:3 * N_TOTAL, 1], (0, pad0)),
        jnp.pad(xf[3 * N_TOTAL:, 1], (0, pad1)),
    ])

    rsqrt_tbl = lax.rsqrt(jnp.maximum(
        jnp.arange(TBL, dtype=jnp.float32), 1.0))
    out0, out1 = _sc_scatter_kernel()(ei4, x0_in, x1_in, rsqrt_tbl)
    o0 = jnp.concatenate([out0[:3 * N_TOTAL],
                          out0[NN_PAD:NN_PAD + 2 * N_TOTAL]])
    o1 = jnp.concatenate([out1[:3 * N_TOTAL],
                          out1[NN_PAD:NN_PAD + 2 * N_TOTAL]])
    o0p = jnp.pad(o0.reshape(S_IN, T, NUM_NODES), ((0, 0), (0, 0), (0, 2)))
    o1p = jnp.pad(o1.reshape(S_IN, T, NUM_NODES), ((0, 0), (0, 0), (0, 2)))

    K = GCN_OUT * NUM_NODES
    eye30 = jnp.eye(NUM_NODES, dtype=jnp.float32)
    P0 = jnp.kron(eye30, gcn_W[0:1, :])
    P1 = jnp.kron(eye30, gcn_W[1:2, :])
    bb = jnp.tile(gcn_b, NUM_NODES)[None, :]
    zrow2 = jnp.zeros((2, K), jnp.float32)
    zrow7 = jnp.zeros((7, K), jnp.float32)
    P = jnp.concatenate([P0, zrow2, P1, zrow2, bb, zrow7])
    U = _fold_u(P, W_ih)

    bsum = (b_ih + b_hh)[None, :]
    k60 = jnp.arange(NUM_IN * NUM_NODES)[:, None]
    n32 = jnp.arange(NODE_PAD)[None, :]
    Sx = ((k60 == 2 * n32) & (n32 < NUM_NODES)).astype(jnp.float32)
    Sy = ((k60 == 2 * n32 + 1) & (n32 < NUM_NODES)).astype(jnp.float32)
    kk = jnp.arange(NODE_PAD * NODE_PAD)
    rep = (kk // NODE_PAD)[None, :]
    til = (kk % NODE_PAD)[None, :]
    n32c = jnp.arange(NODE_PAD)[:, None]
    Rm = (n32c == rep).astype(jnp.float32)
    Qm = (n32c == til).astype(jnp.float32)
    RT = Rm.T
    eyep = (rep == til).astype(jnp.float32)

    fi0 = feature_input[0].reshape(T, NUM_IN * NUM_NODES)
    enc_o, dec_o = _seq_run(fi0, o0p, o1p, U, W_hh, bsum,
                            W1, b1[None, :], W2, b2[None, :], W3, b3[None, :],
                            W4, b4[None, :], Sx, Sy, Rm, Qm, RT, eyep, stats)

    enc = enc_o.reshape(S_IN, N_TOTAL, NUM_IN)
    dec = dec_o.reshape(S_OUT, N_TOTAL, NUM_IN)
    return enc, dec


# speedup vs baseline: 1.0664x; 1.0011x over previous
import functools

import jax
import jax.numpy as jnp
from jax import lax
from jax.experimental import pallas as pl
from jax.experimental.pallas import tpu as pltpu
from jax.experimental.pallas import tpu_sc as plsc

NUM_NODES = 30
NUM_IN = 2
GCN_OUT = 64
HID = 512
T = 128
S_IN = 5
S_OUT = 3
N_TOTAL = T * NUM_NODES
N_EDGES = 32768

NN_PAD = 11776
SLICE = NN_PAD // 16
E_CORE = 3 * N_EDGES
E_TILE = E_CORE // 16
CHUNK = 128
NCHUNK = E_TILE // CHUNK
DUMMY_DST = 11520
NODE_PAD = 32


TBL = 32776


def _sc_body(ei_hbm, x0_hbm, x1_hbm, tbl_hbm, out0_hbm, out1_hbm,
             src_v, dst_v, x0_v, x1_v,
             vals0_v, vals1_v, vals0b_v, vals1b_v,
             vals0c_v, vals1c_v, vals0d_v, vals1d_v,
             degs_v, dinvs_v, s0_v, s1_v, ones_v, table_v,
             sem_s, sem_d, sem_e0, sem_e1, sem_x0, sem_x1, sem_t,
             deg_sh, xs0_sh, xs1_sh, out0_sh, out1_sh):
    vbufs0 = (vals0_v, vals0b_v, vals0c_v, vals0d_v)
    vbufs1 = (vals1_v, vals1b_v, vals1c_v, vals1d_v)
    c = lax.axis_index("c")
    s = lax.axis_index("s")
    base = s * SLICE
    nch = jnp.where(c == 0, NCHUNK, (2 * N_EDGES) // (16 * CHUNK))

    for sl in range(3):
        def _stage(sl=sl):
            st = 3 * c + sl
            pltpu.async_copy(ei_hbm.at[st, 0, pl.ds(s * 16, 16)],
                             src_v.at[pl.ds(sl * 16, 16)], sem_s)
            pltpu.async_copy(ei_hbm.at[st, 1, pl.ds(s * 16, 16)],
                             dst_v.at[pl.ds(sl * 16, 16)], sem_s)
        if sl < 2:
            _stage()
        else:
            pl.when(c == 0)(_stage)

    def _sdrain(j, carry):
        pltpu.make_async_copy(ei_hbm.at[0, 0, pl.ds(s * 16, 16)],
                              src_v.at[pl.ds(0, 16)], sem_s).wait()
        return carry
    lax.fori_loop(0, jnp.where(c == 0, 6, 4), _sdrain, 0)
    for sl in (1, 2):
        off = jnp.full((16,), sl * N_TOTAL, jnp.int32)
        for r in range(16):
            for k in range(CHUNK // 16):
                row = sl * 16 + r
                src_v[row, pl.ds(k * 16, 16)] = (
                    src_v[row, pl.ds(k * 16, 16)] + off)
                dst_v[row, pl.ds(k * 16, 16)] = (
                    dst_v[row, pl.ds(k * 16, 16)] + off)
    pltpu.async_copy(x0_hbm.at[pl.ds(c * NN_PAD + base, SLICE)], s0_v, sem_x0)
    pltpu.async_copy(x1_hbm.at[pl.ds(c * NN_PAD + base, SLICE)], s1_v, sem_x1)
    pltpu.async_copy(tbl_hbm, table_v, sem_t)

    for i in range(CHUNK // 16):
        ones_v[pl.ds(i * 16, 16)] = jnp.full((16,), 1.0, jnp.float32)
    for i in range(SLICE // 16):
        degs_v[pl.ds(i * 16, 16)] = jnp.full((16,), 0.0, jnp.float32)
    pltpu.sync_copy(degs_v, deg_sh.at[pl.ds(base, SLICE)])
    plsc.subcore_barrier()

    def deg_step(j, carry):
        @pl.when(j >= 4)
        def _w():
            pltpu.make_async_copy(ones_v, deg_sh.at[dst_v.at[j]],
                                  sem_d).wait()
        pltpu.async_copy(ones_v, deg_sh.at[dst_v.at[j]], sem_d, add=True)
        return carry
    lax.fori_loop(0, nch, deg_step, 0)
    for _ in range(4):
        pltpu.make_async_copy(ones_v, deg_sh.at[dst_v.at[0]], sem_d).wait()
    plsc.subcore_barrier()

    pltpu.make_async_copy(x0_hbm.at[pl.ds(c * NN_PAD + base, SLICE)],
                          s0_v, sem_x0).wait()
    pltpu.make_async_copy(x1_hbm.at[pl.ds(c * NN_PAD + base, SLICE)],
                          s1_v, sem_x1).wait()
    pltpu.make_async_copy(tbl_hbm, table_v, sem_t).wait()
    pltpu.sync_copy(deg_sh.at[pl.ds(base, SLICE)], degs_v)
    for i in range(SLICE // 16):
        d = degs_v[pl.ds(i * 16, 16)] + 1.0
        y = plsc.load_gather(table_v, [d.astype(jnp.int32)])
        dinvs_v[pl.ds(i * 16, 16)] = y
        s0_v[pl.ds(i * 16, 16)] = s0_v[pl.ds(i * 16, 16)] * y
        s1_v[pl.ds(i * 16, 16)] = s1_v[pl.ds(i * 16, 16)] * y
    pltpu.sync_copy(s0_v, xs0_sh.at[pl.ds(base, SLICE)])
    pltpu.sync_copy(s1_v, xs1_sh.at[pl.ds(base, SLICE)])
    pltpu.sync_copy(s0_v, out0_sh.at[pl.ds(base, SLICE)])
    pltpu.sync_copy(s1_v, out1_sh.at[pl.ds(base, SLICE)])
    plsc.subcore_barrier()

    pltpu.sync_copy(xs0_sh, x0_v)
    pltpu.sync_copy(xs1_sh, x1_v)

    def edge_step(j, carry):
        @pl.when(j >= 4)
        def _w():
            pltpu.make_async_copy(vals0_v, out0_sh.at[dst_v.at[j]],
                                  sem_e0).wait()
            pltpu.make_async_copy(vals1_v, out1_sh.at[dst_v.at[j]],
                                  sem_e1).wait()

        def run(b0, b1):
            for k in range(CHUNK // 16):
                src = src_v[j, pl.ds(k * 16, 16)]
                b0[pl.ds(k * 16, 16)] = plsc.load_gather(x0_v, [src])
                b1[pl.ds(k * 16, 16)] = plsc.load_gather(x1_v, [src])
            pltpu.async_copy(b0, out0_sh.at[dst_v.at[j]], sem_e0, add=True)
            pltpu.async_copy(b1, out1_sh.at[dst_v.at[j]], sem_e1, add=True)

        for par in range(4):
            pl.when(j % 4 == par)(
                functools.partial(run, vbufs0[par], vbufs1[par]))
        return carry
    lax.fori_loop(0, nch, edge_step, 0)
    for _ in range(4):
        pltpu.make_async_copy(vals0_v, out0_sh.at[dst_v.at[0]],
                              sem_e0).wait()
        pltpu.make_async_copy(vals1_v, out1_sh.at[dst_v.at[0]],
                              sem_e1).wait()
    plsc.subcore_barrier()

    pltpu.sync_copy(out0_sh.at[pl.ds(base, SLICE)], s0_v)
    pltpu.sync_copy(out1_sh.at[pl.ds(base, SLICE)], s1_v)
    for i in range(SLICE // 16):
        y = dinvs_v[pl.ds(i * 16, 16)]
        s0_v[pl.ds(i * 16, 16)] = s0_v[pl.ds(i * 16, 16)] * y
        s1_v[pl.ds(i * 16, 16)] = s1_v[pl.ds(i * 16, 16)] * y
    pltpu.sync_copy(s0_v, out0_hbm.at[pl.ds(c * NN_PAD + base, SLICE)])
    pltpu.sync_copy(s1_v, out1_hbm.at[pl.ds(c * NN_PAD + base, SLICE)])


@functools.cache
def _sc_scatter_kernel():
    return functools.partial(
        pl.kernel,
        out_type=[jax.ShapeDtypeStruct((2 * NN_PAD,), jnp.float32),
                  jax.ShapeDtypeStruct((2 * NN_PAD,), jnp.float32)],
        mesh=plsc.VectorSubcoreMesh(core_axis_name="c", subcore_axis_name="s",
                                    num_cores=2, num_subcores=16),
        compiler_params=pltpu.CompilerParams(needs_layout_passes=False),
        scratch_types=[
        pltpu.VMEM((NCHUNK, CHUNK), jnp.int32),
        pltpu.VMEM((NCHUNK, CHUNK), jnp.int32),
        pltpu.VMEM((NN_PAD,), jnp.float32),
        pltpu.VMEM((NN_PAD,), jnp.float32),
        pltpu.VMEM((CHUNK,), jnp.float32),
        pltpu.VMEM((CHUNK,), jnp.float32),
        pltpu.VMEM((CHUNK,), jnp.float32),
        pltpu.VMEM((CHUNK,), jnp.float32),
        pltpu.VMEM((CHUNK,), jnp.float32),
        pltpu.VMEM((CHUNK,), jnp.float32),
        pltpu.VMEM((CHUNK,), jnp.float32),
        pltpu.VMEM((CHUNK,), jnp.float32),
        pltpu.VMEM((SLICE,), jnp.float32),
        pltpu.VMEM((SLICE,), jnp.float32),
        pltpu.VMEM((SLICE,), jnp.float32),
        pltpu.VMEM((SLICE,), jnp.float32),
        pltpu.VMEM((CHUNK,), jnp.float32),
        pltpu.VMEM((TBL,), jnp.float32),
        pltpu.SemaphoreType.DMA,
        pltpu.SemaphoreType.DMA,
        pltpu.SemaphoreType.DMA,
        pltpu.SemaphoreType.DMA,
        pltpu.SemaphoreType.DMA,
        pltpu.SemaphoreType.DMA,
        pltpu.SemaphoreType.DMA,
        pltpu.VMEM_SHARED((NN_PAD,), jnp.float32),
        pltpu.VMEM_SHARED((NN_PAD,), jnp.float32),
        pltpu.VMEM_SHARED((NN_PAD,), jnp.float32),
        pltpu.VMEM_SHARED((NN_PAD,), jnp.float32),
        pltpu.VMEM_SHARED((NN_PAD,), jnp.float32),
        ],
    )(_sc_body)


def _fold_body(p_ref, w_ref, u_ref):
    u_ref[...] = jnp.dot(p_ref[...], w_ref[...],
                         preferred_element_type=jnp.float32)


def _fold_u(P, W_ih):
    n_blk = 8
    blk = (4 * HID) // n_blk
    return pl.pallas_call(
        _fold_body,
        grid=(n_blk,),
        in_specs=[
            pl.BlockSpec((72, GCN_OUT * NUM_NODES), lambda n: (0, 0)),
            pl.BlockSpec((GCN_OUT * NUM_NODES, blk), lambda n: (0, n)),
        ],
        out_specs=pl.BlockSpec((72, blk), lambda n: (0, n)),
        out_shape=jax.ShapeDtypeStruct((72, 4 * HID), jnp.float32),
    )(P, W_ih)


def _seq_body(fi0_ref, o0_ref, o1_ref, u_ref, whh_ref, bsum_ref,
              w1_ref, b1_ref, w2_ref, b2_ref, w3_ref, b3_ref,
              w4_ref, b4_ref, sx_ref, sy_ref, rm_ref, qm_ref, rt_ref,
              eyep_ref, stats_ref, enc_ref, dec_ref):
    enc_ref[0] = fi0_ref[...]
    U0 = u_ref[0:32, :]
    U1 = u_ref[32:64, :]
    bvec = u_ref[64:65, :]
    bias = bvec + bsum_ref[...]
    std0 = stats_ref[0:1, 0:1]
    std1 = stats_ref[0:1, 1:2]
    mean0 = stats_ref[1:2, 0:1]
    mean1 = stats_ref[1:2, 1:2]

    h = jnp.zeros((T, HID), jnp.float32)
    c = jnp.zeros((T, HID), jnp.float32)
    p = None
    for step in range(S_IN - 1 + S_OUT):
        if step < S_IN:
            m0 = o0_ref[step]
            m1 = o1_ref[step]
        else:
            def mm(a, b):
                return jnp.dot(a, b, preferred_element_type=jnp.float32)
            pxs = mm(p, sx_ref[...])
            pys = mm(p, sy_ref[...])
            fx = pxs * std0 + mean0
            fy = pys * std1 + mean1
            col = lax.broadcasted_iota(jnp.int32, (T, NODE_PAD), 1)
            exn = jnp.where((fx > 0.04) & (fy > 0.04) & (col < NUM_NODES),
                            1.0, 0.0)
            Rm = rm_ref[...]
            Qm = qm_ref[...]
            fxr = mm(pxs, Rm) * std0 + mean0
            fxt = mm(pxs, Qm) * std0 + mean0
            fyr = mm(pys, Rm) * std1 + mean1
            fyt = mm(pys, Qm) * std1 + mean1
            dx = fxr - fxt
            dy = fyr - fyt
            d2 = dx * dx + dy * dy
            cond = jnp.where((d2 > 0.0) & (d2 < 100.0), 1.0, 0.0)
            ep = mm(exn, Rm) * mm(exn, Qm)
            eyep = eyep_ref[...]
            A = eyep + (1.0 - eyep) * (ep * cond)
            deg = mm(A, rt_ref[...])
            dinv = lax.rsqrt(jnp.maximum(deg, 1e-12))
            W = A * (mm(dinv, Rm) * mm(dinv, Qm))
            m0 = mm(W * mm(pxs, Qm), rt_ref[...])
            m1 = mm(W * mm(pys, Qm), rt_ref[...])
        gates = (jnp.dot(m0, U0, preferred_element_type=jnp.float32)
                 + jnp.dot(m1, U1, preferred_element_type=jnp.float32)
                 + jnp.dot(h, whh_ref[...], preferred_element_type=jnp.float32)
                 + bias)
        i_g = jax.nn.sigmoid(gates[:, 0:HID])
        f_g = jax.nn.sigmoid(gates[:, HID:2 * HID])
        g_g = jnp.tanh(gates[:, 2 * HID:3 * HID])
        o_g = jax.nn.sigmoid(gates[:, 3 * HID:4 * HID])
        c = f_g * c + i_g * g_g
        h = o_g * jnp.tanh(c)
        m = jax.nn.relu(jnp.dot(h, w1_ref[...],
                                preferred_element_type=jnp.float32)
                        + b1_ref[...])
        m = jax.nn.relu(jnp.dot(m, w2_ref[...],
                                preferred_element_type=jnp.float32)
                        + b2_ref[...])
        m = jax.nn.relu(jnp.dot(m, w3_ref[...],
                                preferred_element_type=jnp.float32)
                        + b3_ref[...])
        p = jnp.dot(m, w4_ref[...],
                    preferred_element_type=jnp.float32) + b4_ref[...]
        if step < S_IN - 1:
            enc_ref[step + 1] = p
        else:
            dec_ref[step - (S_IN - 1)] = p


def _seq_run(fi0, o0p, o1p, U, W_hh, bsum, W1, b1, W2, b2, W3, b3, W4, b4,
             Sx, Sy, Rm, Qm, RT, eyep, stats):
    return pl.pallas_call(
        _seq_body,
        out_shape=[
            jax.ShapeDtypeStruct((S_IN, T, NUM_IN * NUM_NODES), jnp.float32),
            jax.ShapeDtypeStruct((S_OUT, T, NUM_IN * NUM_NODES), jnp.float32),
        ],
    )(fi0, o0p, o1p, U, W_hh, bsum, W1, b1, W2, b2, W3, b3, W4, b4,
      Sx, Sy, Rm, Qm, RT, eyep, stats)


def kernel(feature_input, edge_index, batch_index, number_of_trajectories,
           stats, gcn_W, gcn_b, W_ih, W_hh, b_ih, b_hh,
           W1, b1, W2, b2, W3, b3, W4, b4):
    ei4 = edge_index.astype(jnp.int32).reshape(S_IN, 2, N_EDGES // CHUNK,
                                               CHUNK)

    xf = feature_input.reshape(S_IN * N_TOTAL, NUM_IN)
    pad0 = NN_PAD - 3 * N_TOTAL
    pad1 = NN_PAD - 2 * N_TOTAL
    x0_in = jnp.concatenate([
        jnp.pad(xf[:3 * N_TOTAL, 0], (0, pad0)),
        jnp.pad(xf[3 * N_TOTAL:, 0], (0, pad1)),
    ])
    x1_in = jnp.concatenate([
        jnp.pad(xf[:3 * N_TOTAL, 1], (0, pad0)),
        jnp.pad(xf[3 * N_TOTAL:, 1], (0, pad1)),
    ])

    rsqrt_tbl = lax.rsqrt(jnp.maximum(
        jnp.arange(TBL, dtype=jnp.float32), 1.0))
    out0, out1 = _sc_scatter_kernel()(ei4, x0_in, x1_in, rsqrt_tbl)
    o0 = jnp.concatenate([out0[:3 * N_TOTAL],
                          out0[NN_PAD:NN_PAD + 2 * N_TOTAL]])
    o1 = jnp.concatenate([out1[:3 * N_TOTAL],
                          out1[NN_PAD:NN_PAD + 2 * N_TOTAL]])
    o0p = jnp.pad(o0.reshape(S_IN, T, NUM_NODES), ((0, 0), (0, 0), (0, 2)))
    o1p = jnp.pad(o1.reshape(S_IN, T, NUM_NODES), ((0, 0), (0, 0), (0, 2)))

    K = GCN_OUT * NUM_NODES
    eye30 = jnp.eye(NUM_NODES, dtype=jnp.float32)
    P0 = jnp.kron(eye30, gcn_W[0:1, :])
    P1 = jnp.kron(eye30, gcn_W[1:2, :])
    bb = jnp.tile(gcn_b, NUM_NODES)[None, :]
    zrow2 = jnp.zeros((2, K), jnp.float32)
    zrow7 = jnp.zeros((7, K), jnp.float32)
    P = jnp.concatenate([P0, zrow2, P1, zrow2, bb, zrow7])
    U = _fold_u(P, W_ih)

    bsum = (b_ih + b_hh)[None, :]
    k60 = jnp.arange(NUM_IN * NUM_NODES)[:, None]
    n32 = jnp.arange(NODE_PAD)[None, :]
    Sx = ((k60 == 2 * n32) & (n32 < NUM_NODES)).astype(jnp.float32)
    Sy = ((k60 == 2 * n32 + 1) & (n32 < NUM_NODES)).astype(jnp.float32)
    kk = jnp.arange(NODE_PAD * NODE_PAD)
    rep = (kk // NODE_PAD)[None, :]
    til = (kk % NODE_PAD)[None, :]
    n32c = jnp.arange(NODE_PAD)[:, None]
    Rm = (n32c == rep).astype(jnp.float32)
    Qm = (n32c == til).astype(jnp.float32)
    RT = Rm.T
    eyep = (rep == til).astype(jnp.float32)

    fi0 = feature_input[0].reshape(T, NUM_IN * NUM_NODES)
    enc_o, dec_o = _seq_run(fi0, o0p, o1p, U, W_hh, bsum,
                            W1, b1[None, :], W2, b2[None, :], W3, b3[None, :],
                            W4, b4[None, :], Sx, Sy, Rm, Qm, RT, eyep, stats)

    enc = enc_o.reshape(S_IN, N_TOTAL, NUM_IN)
    dec = dec_o.reshape(S_OUT, N_TOTAL, NUM_IN)
    return enc, dec
